# baseline (device time: 32214 ns/iter reference)
import jax
import jax.numpy as jnp
from jax import lax
from jax.experimental import pallas as pl
from jax.experimental.pallas import tpu as pltpu

N_DEV = 8
N_GLOBAL = 8192
EPS = 1e-5
M = 4096
CI = 512
N_IN = M // CI
BM = 1024
NB = M // BM


def _partials_call(x):

    def body(x_ref, p_ref, xv_ref, in_sems):
        in_dmas = []
        for c in range(N_IN):
            rows = pl.ds(c * CI, CI)
            dma = pltpu.make_async_copy(
                x_ref.at[rows, :], xv_ref.at[rows, :], in_sems.at[c])
            dma.start()
            in_dmas.append(dma)
        for c in range(N_IN):
            in_dmas[c].wait()
            a = xv_ref[pl.ds(c * CI, CI), :]
            st = jnp.stack([jnp.sum(a, axis=1), jnp.sum(a * a, axis=1)])
            p_ref[:, c * CI:(c + 1) * CI] = st

    return pl.pallas_call(
        body,
        out_shape=jax.ShapeDtypeStruct((2, M), jnp.float32),
        in_specs=[pl.BlockSpec(memory_space=pl.ANY)],
        out_specs=pl.BlockSpec(memory_space=pltpu.VMEM),
        scratch_shapes=[
            pltpu.VMEM((M, 1024), jnp.float32),
            pltpu.SemaphoreType.DMA((N_IN,)),
        ],
        compiler_params=pltpu.CompilerParams(
            vmem_limit_bytes=60 * 1024 * 1024,
        ),
    )(x)


def _allreduce_call(partials):

    def body(p_ref, mr_ref, comm_ref, send_sems, recv_sems):
        my_pos = lax.axis_index("i")

        comm_ref[my_pos] = p_ref[...]

        barrier_sem = pltpu.get_barrier_semaphore()
        for k in range(1, N_DEV):
            pl.semaphore_signal(
                barrier_sem, inc=1,
                device_id=((my_pos + k) % N_DEV,),
                device_id_type=pl.DeviceIdType.MESH,
            )
        pl.semaphore_wait(barrier_sem, N_DEV - 1)

        sends = []
        for k in range(1, N_DEV):
            peer = (my_pos + k) % N_DEV
            rdma = pltpu.make_async_remote_copy(
                src_ref=comm_ref.at[my_pos],
                dst_ref=comm_ref.at[my_pos],
                send_sem=send_sems.at[k],
                recv_sem=recv_sems.at[my_pos],
                device_id=(peer,),
                device_id_type=pl.DeviceIdType.MESH,
            )
            rdma.start()
            sends.append(rdma)
        for k in range(1, N_DEV):
            src = (my_pos + k) % N_DEV
            pltpu.make_async_remote_copy(
                src_ref=comm_ref.at[src],
                dst_ref=comm_ref.at[src],
                send_sem=send_sems.at[k],
                recv_sem=recv_sems.at[src],
                device_id=(src,),
                device_id_type=pl.DeviceIdType.MESH,
            ).wait_recv()

        totals = jnp.sum(comm_ref[...], axis=0)
        t = jnp.transpose(totals)
        mean = t[:, 0:1] / N_GLOBAL
        var = t[:, 1:2] / N_GLOBAL - mean * mean
        mr_ref[:, 0:1] = mean
        mr_ref[:, 1:2] = lax.rsqrt(var + EPS)

        for rdma in sends:
            rdma.wait_send()

    return pl.pallas_call(
        body,
        out_shape=jax.ShapeDtypeStruct((M, 2), jnp.float32),
        in_specs=[pl.BlockSpec(memory_space=pltpu.VMEM)],
        out_specs=pl.BlockSpec(memory_space=pltpu.VMEM),
        scratch_shapes=[
            pltpu.VMEM((N_DEV, 2, M), jnp.float32),
            pltpu.SemaphoreType.DMA((N_DEV,)),
            pltpu.SemaphoreType.DMA((N_DEV,)),
        ],
        compiler_params=pltpu.CompilerParams(
            collective_id=0,
            vmem_limit_bytes=60 * 1024 * 1024,
        ),
    )(partials)


def _normalize_call(x, mr, gamma, beta):
    m, n_per = x.shape

    def body(x_ref, mr_ref, gamma_ref, beta_ref, out_ref):
        a = x_ref[...]
        mean = mr_ref[:, 0:1]
        rstd = mr_ref[:, 1:2]
        out_ref[...] = ((a - mean) * rstd * gamma_ref[...][None, :]
                        + beta_ref[...][None, :])

    return pl.pallas_call(
        body,
        grid=(NB,),
        out_shape=jax.ShapeDtypeStruct((m, n_per), jnp.float32),
        in_specs=[
            pl.BlockSpec((BM, n_per), lambda s: (s, 0)),
            pl.BlockSpec((BM, 2), lambda s: (s, 0)),
            pl.BlockSpec((n_per,), lambda s: (0,)),
            pl.BlockSpec((n_per,), lambda s: (0,)),
        ],
        out_specs=pl.BlockSpec((BM, n_per), lambda s: (s, 0)),
        compiler_params=pltpu.CompilerParams(
            dimension_semantics=("parallel",),
            vmem_limit_bytes=60 * 1024 * 1024,
        ),
    )(x, mr, gamma, beta)


def kernel(x, gamma, beta):
    partials = _partials_call(x)
    mr = _allreduce_call(partials)
    return _normalize_call(x, mr, gamma, beta)


# device time: 28747 ns/iter; 1.1206x vs baseline; 1.1206x over previous
import jax
import jax.numpy as jnp
from jax import lax
from jax.experimental import pallas as pl
from jax.experimental.pallas import tpu as pltpu

N_DEV = 8
N_GLOBAL = 8192
EPS = 1e-5
M = 4096
H = M // 2
CI = 512
N_IN = M // CI
BM = 1024
NB = M // BM


def _stats_call(x):

    def body(x_ref, mr_ref, xv_ref, comm_ref, in_sems, send_sems, recv_sems):
        my_pos = lax.axis_index("i")

        in_dmas = []
        for c in range(N_IN):
            rows = pl.ds(c * CI, CI)
            dma = pltpu.make_async_copy(
                x_ref.at[rows, :], xv_ref.at[rows, :], in_sems.at[c])
            dma.start()
            in_dmas.append(dma)

        barrier_sem = pltpu.get_barrier_semaphore()
        for k in range(1, N_DEV):
            pl.semaphore_signal(
                barrier_sem, inc=1,
                device_id=((my_pos + k) % N_DEV,),
                device_id_type=pl.DeviceIdType.MESH,
            )

        def partials_chunk(c):
            in_dmas[c].wait()
            a = xv_ref[pl.ds(c * CI, CI), :]
            st = jnp.stack([jnp.sum(a, axis=1), jnp.sum(a * a, axis=1)])
            off = (c % (N_IN // 2)) * CI
            comm_ref[(c // (N_IN // 2)) * N_DEV + my_pos, :, off:off + CI] = st

        def send_half(h):
            sends = []
            for k in range(1, N_DEV):
                peer = (my_pos + k) % N_DEV
                rdma = pltpu.make_async_remote_copy(
                    src_ref=comm_ref.at[h * N_DEV + my_pos],
                    dst_ref=comm_ref.at[h * N_DEV + my_pos],
                    send_sem=send_sems.at[h * N_DEV + k],
                    recv_sem=recv_sems.at[h * N_DEV + my_pos],
                    device_id=(peer,),
                    device_id_type=pl.DeviceIdType.MESH,
                )
                rdma.start()
                sends.append(rdma)
            return sends

        def wait_half(h):
            for k in range(1, N_DEV):
                src = (my_pos + k) % N_DEV
                pltpu.make_async_remote_copy(
                    src_ref=comm_ref.at[h * N_DEV + src],
                    dst_ref=comm_ref.at[h * N_DEV + src],
                    send_sem=send_sems.at[h * N_DEV + k],
                    recv_sem=recv_sems.at[h * N_DEV + src],
                    device_id=(src,),
                    device_id_type=pl.DeviceIdType.MESH,
                ).wait_recv()
            totals = jnp.sum(
                comm_ref[pl.ds(h * N_DEV, N_DEV)], axis=0)
            t = jnp.transpose(totals)
            mean = t[:, 0:1] / N_GLOBAL
            var = t[:, 1:2] / N_GLOBAL - mean * mean
            mr_ref[pl.ds(h * H, H), 0:1] = mean
            mr_ref[pl.ds(h * H, H), 1:2] = lax.rsqrt(var + EPS)

        for c in range(N_IN // 2):
            partials_chunk(c)
        pl.semaphore_wait(barrier_sem, N_DEV - 1)
        sends0 = send_half(0)
        for c in range(N_IN // 2, N_IN):
            partials_chunk(c)
        sends1 = send_half(1)
        wait_half(0)
        wait_half(1)
        for rdma in sends0 + sends1:
            rdma.wait_send()

    return pl.pallas_call(
        body,
        out_shape=jax.ShapeDtypeStruct((M, 2), jnp.float32),
        in_specs=[pl.BlockSpec(memory_space=pl.ANY)],
        out_specs=pl.BlockSpec(memory_space=pltpu.VMEM),
        scratch_shapes=[
            pltpu.VMEM((M, 1024), jnp.float32),
            pltpu.VMEM((2 * N_DEV, 2, H), jnp.float32),
            pltpu.SemaphoreType.DMA((N_IN,)),
            pltpu.SemaphoreType.DMA((2 * N_DEV,)),
            pltpu.SemaphoreType.DMA((2 * N_DEV,)),
        ],
        compiler_params=pltpu.CompilerParams(
            collective_id=0,
            vmem_limit_bytes=60 * 1024 * 1024,
        ),
    )(x)


def _normalize_call(x, mr, gamma, beta):
    m, n_per = x.shape

    def body(x_ref, mr_ref, gamma_ref, beta_ref, out_ref):
        a = x_ref[...]
        mean = mr_ref[:, 0:1]
        rstd = mr_ref[:, 1:2]
        out_ref[...] = ((a - mean) * rstd * gamma_ref[...][None, :]
                        + beta_ref[...][None, :])

    return pl.pallas_call(
        body,
        grid=(NB,),
        out_shape=jax.ShapeDtypeStruct((m, n_per), jnp.float32),
        in_specs=[
            pl.BlockSpec((BM, n_per), lambda s: (s, 0)),
            pl.BlockSpec((BM, 2), lambda s: (s, 0)),
            pl.BlockSpec((n_per,), lambda s: (0,)),
            pl.BlockSpec((n_per,), lambda s: (0,)),
        ],
        out_specs=pl.BlockSpec((BM, n_per), lambda s: (s, 0)),
        compiler_params=pltpu.CompilerParams(
            dimension_semantics=("parallel",),
            vmem_limit_bytes=60 * 1024 * 1024,
        ),
    )(x, mr, gamma, beta)


def kernel(x, gamma, beta):
    mr = _stats_call(x)
    return _normalize_call(x, mr, gamma, beta)
